# baseline (speedup 1.0000x reference)
import functools

import jax
import jax.numpy as jnp
from jax import lax
from jax.experimental import pallas as pl
from jax.experimental.pallas import tpu as pltpu
from jax.experimental.pallas import tpu_sc as plsc

_SMALL = 1e-08
_NC = 2
_NS = 16
_LANES = 16


def _build_table(node_states, Wt, b2, T, N, D, BN):
    def body(x_ref, wt_ref, b_ref, out_ref):
        mm = jnp.dot(x_ref[...], wt_ref[...], preferred_element_type=jnp.float32)
        out_ref[...] = mm + b_ref[0]

    return pl.pallas_call(
        body,
        grid=(N // BN, T),
        in_specs=[
            pl.BlockSpec((BN, D), lambda i, t: (i, 0)),
            pl.BlockSpec((D, D), lambda i, t: (0, t)),
            pl.BlockSpec((1, 1, D), lambda i, t: (t, 0, 0)),
        ],
        out_specs=pl.BlockSpec((BN, D), lambda i, t: (t * (N // BN) + i, 0)),
        out_shape=jax.ShapeDtypeStruct((T * N, D), jnp.float32),
    )(node_states, Wt, b2)


def _edge_scatter(table, pairs, N, D, E, M):
    NW = _NC * _NS
    EPW = E // NW
    CH = 80
    NCH = EPW // CH
    BCH = 25
    NB = NCH // BCH
    EPB = BCH * CH
    RPT = N // _NS
    CW = 16

    mesh = plsc.VectorSubcoreMesh(core_axis_name="c", subcore_axis_name="s")

    @functools.partial(
        pl.kernel,
        out_type=(jax.ShapeDtypeStruct((_NC, N, D), jnp.float32),
                  jax.ShapeDtypeStruct((_NC, N, CW), jnp.float32)),
        mesh=mesh,
        scratch_types=[
            pltpu.VMEM_SHARED((N, D), jnp.float32),
            pltpu.VMEM_SHARED((N, CW), jnp.float32),
            pltpu.VMEM((EPB, 2), jnp.int32),
            pltpu.VMEM((CH,), jnp.int32),
            pltpu.VMEM((CH,), jnp.int32),
            pltpu.VMEM((CH,), jnp.int32),
            pltpu.VMEM((CH,), jnp.int32),
            pltpu.VMEM((CH, D), jnp.float32),
            pltpu.VMEM((CH, D), jnp.float32),
            pltpu.VMEM((CH, CW), jnp.float32),
            pltpu.SemaphoreType.DMA,
            pltpu.SemaphoreType.DMA,
            pltpu.SemaphoreType.DMA,
        ],
        compiler_params=pltpu.CompilerParams(use_tc_tiling_on_sc=False,
                                             needs_layout_passes=False),
    )
    def run(pairs_hbm, table_hbm, out_hbm, cnt_hbm, acc, cacc, pbuf,
            sidx0, sidx1, didx0, didx1, rows0, rows1, ones,
            sem0, sem1, semc):
        cid = lax.axis_index("c")
        sid = lax.axis_index("s")
        wid = cid * _NS + sid
        row0 = sid * RPT
        rows = (rows0, rows1)
        sidx = (sidx0, sidx1)
        didx = (didx0, didx1)
        sems = (sem0, sem1)
        toff = (wid * EPW) // M * N

        for b in range(2):
            @pl.loop(0, CH)
            def _(r):
                @pl.loop(0, D, step=_LANES)
                def _(c):
                    rows[b][r, pl.ds(c, _LANES)] = jnp.zeros((_LANES,),
                                                             jnp.float32)

        @pl.loop(0, CH)
        def _(r):
            ones[r, pl.ds(0, CW)] = jnp.zeros((CW,), jnp.float32)

        @pl.loop(0, RPT - 65, step=CH)
        def _(r):
            pltpu.sync_copy(rows0, acc.at[pl.ds(row0 + r, CH)])
        pltpu.sync_copy(rows1.at[pl.ds(0, 65)],
                        acc.at[pl.ds(row0 + RPT - 65, 65)])

        @pl.loop(0, RPT - 65, step=CH)
        def _(r):
            pltpu.sync_copy(ones, cacc.at[pl.ds(row0 + r, CH)])
        pltpu.sync_copy(ones.at[pl.ds(0, 65)],
                        cacc.at[pl.ds(row0 + RPT - 65, 65)])

        @pl.loop(0, CH)
        def _(r):
            ones[r, pl.ds(0, CW)] = jnp.ones((CW,), jnp.float32)

        plsc.subcore_barrier()

        lane = lax.iota(jnp.int32, _LANES)
        zl = jnp.zeros((_LANES,), jnp.int32)
        ol = jnp.ones((_LANES,), jnp.int32)

        def build_idx(j, b):
            base = j * CH
            for k in range(CH // _LANES):
                rvec = base + k * _LANES + lane
                s = plsc.load_gather(pbuf, [rvec, zl])
                d = plsc.load_gather(pbuf, [rvec, ol])
                sidx[b][pl.ds(k * _LANES, _LANES)] = s + toff
                didx[b][pl.ds(k * _LANES, _LANES)] = d

        def start_gather(b):
            pltpu.async_copy(table_hbm.at[sidx[b]], rows[b], sems[b])

        def wait_gather(b):
            pltpu.make_async_copy(table_hbm.at[sidx[b]], rows[b],
                                  sems[b]).wait()

        def scatter(b):
            pltpu.async_copy(ones, cacc.at[didx[b]], semc, add=True)
            pltpu.sync_copy(rows[b], acc.at[didx[b]], add=True)
            pltpu.make_async_copy(ones, cacc.at[didx[b]], semc).wait()

        @pl.loop(0, NB)
        def _(blk):
            pltpu.sync_copy(pairs_hbm.at[pl.ds(wid * EPW + blk * EPB, EPB)],
                            pbuf)

            build_idx(0, 0)
            start_gather(0)

            @pl.loop(0, (BCH - 1) // 2)
            def _(i):
                j = 2 * i
                build_idx(j + 1, 1)
                start_gather(1)
                wait_gather(0)
                scatter(0)
                build_idx(j + 2, 0)
                start_gather(0)
                wait_gather(1)
                scatter(1)

            wait_gather(0)
            scatter(0)

        plsc.subcore_barrier()

        pltpu.sync_copy(acc.at[pl.ds(row0, RPT)],
                        out_hbm.at[cid, pl.ds(row0, RPT)])
        pltpu.sync_copy(cacc.at[pl.ds(row0, RPT)],
                        cnt_hbm.at[cid, pl.ds(row0, RPT)])

    return run(pairs, table)


def _normalize(partials, counts, N, D, CW, BN):
    def body(p_ref, c_ref, o_ref):
        s = p_ref[0] + p_ref[1]
        cnt = (c_ref[0] + c_ref[1])[:, :1]
        div = jnp.where(cnt == 0.0, 1.0, cnt) + _SMALL
        o_ref[...] = s / div

    return pl.pallas_call(
        body,
        grid=(N // BN,),
        in_specs=[
            pl.BlockSpec((_NC, BN, D), lambda i: (0, i, 0)),
            pl.BlockSpec((_NC, BN, CW), lambda i: (0, i, 0)),
        ],
        out_specs=pl.BlockSpec((BN, D), lambda i: (i, 0)),
        out_shape=jax.ShapeDtypeStruct((N, D), jnp.float32),
    )(partials, counts)


def kernel(edge_lists, node_states, pos_lists, W, b):
    del pos_lists
    N, D = node_states.shape
    T, M, _ = edge_lists.shape
    E = T * M

    pairs = edge_lists.reshape(E, 2)
    Wt = W.T
    b2 = b.reshape(T, 1, D)

    table = _build_table(node_states, Wt, b2, T, N, D, BN=1000)
    partials, counts = _edge_scatter(table, pairs, N, D, E, M)
    return _normalize(partials, counts, N, D, CW=16, BN=1000)


# speedup vs baseline: 1.2824x; 1.2824x over previous
import functools

import jax
import jax.numpy as jnp
from jax import lax
from jax.experimental import pallas as pl
from jax.experimental.pallas import tpu as pltpu
from jax.experimental.pallas import tpu_sc as plsc

_SMALL = 1e-08
_NC = 2
_NS = 16
_LANES = 16


def _build_table(node_states, Wt, b2, T, N, D, BN):
    def body(x_ref, wt_ref, b_ref, out_ref):
        mm = jnp.dot(x_ref[...], wt_ref[...], preferred_element_type=jnp.float32)
        out_ref[...] = mm + b_ref[0]

    return pl.pallas_call(
        body,
        grid=(N // BN, T),
        in_specs=[
            pl.BlockSpec((BN, D), lambda i, t: (i, 0)),
            pl.BlockSpec((D, D), lambda i, t: (0, t)),
            pl.BlockSpec((1, 1, D), lambda i, t: (t, 0, 0)),
        ],
        out_specs=pl.BlockSpec((BN, D), lambda i, t: (t * (N // BN) + i, 0)),
        out_shape=jax.ShapeDtypeStruct((T * N, D), jnp.float32),
    )(node_states, Wt, b2)


def _edge_scatter(table, pairs, N, D, E, M):
    NW = _NC * _NS
    EPW = E // NW
    CH = 80
    NCH = EPW // CH
    BCH = 25
    NB = NCH // BCH
    EPB = BCH * CH
    RPT = N // _NS
    CW = 16

    mesh = plsc.VectorSubcoreMesh(core_axis_name="c", subcore_axis_name="s")

    @functools.partial(
        pl.kernel,
        out_type=(jax.ShapeDtypeStruct((_NC, N, D), jnp.float32),
                  jax.ShapeDtypeStruct((_NC, N, CW), jnp.float32)),
        mesh=mesh,
        scratch_types=[
            pltpu.VMEM_SHARED((N, D), jnp.float32),
            pltpu.VMEM_SHARED((N, CW), jnp.float32),
            pltpu.VMEM((2 * EPB,), jnp.int32),
            pltpu.VMEM((CH,), jnp.int32),
            pltpu.VMEM((CH,), jnp.int32),
            pltpu.VMEM((CH,), jnp.int32),
            pltpu.VMEM((CH,), jnp.int32),
            pltpu.VMEM((CH, D), jnp.float32),
            pltpu.VMEM((CH, D), jnp.float32),
            pltpu.VMEM((CH, CW), jnp.float32),
            pltpu.SemaphoreType.DMA,
            pltpu.SemaphoreType.DMA,
            pltpu.SemaphoreType.DMA,
        ],
        compiler_params=pltpu.CompilerParams(use_tc_tiling_on_sc=False,
                                             needs_layout_passes=False),
    )
    def run(pairs_hbm, table_hbm, out_hbm, cnt_hbm, acc, cacc, pbuf,
            sidx0, sidx1, didx0, didx1, rows0, rows1, ones,
            sem0, sem1, semc):
        cid = lax.axis_index("c")
        sid = lax.axis_index("s")
        wid = cid * _NS + sid
        row0 = sid * RPT
        rows = (rows0, rows1)
        sidx = (sidx0, sidx1)
        didx = (didx0, didx1)
        sems = (sem0, sem1)
        toff = (wid * EPW) // M * N

        for b in range(2):
            @pl.loop(0, CH)
            def _(r):
                @pl.loop(0, D, step=_LANES)
                def _(c):
                    rows[b][r, pl.ds(c, _LANES)] = jnp.zeros((_LANES,),
                                                             jnp.float32)

        @pl.loop(0, CH)
        def _(r):
            ones[r, pl.ds(0, CW)] = jnp.zeros((CW,), jnp.float32)

        @pl.loop(0, RPT - 65, step=CH)
        def _(r):
            pltpu.sync_copy(rows0, acc.at[pl.ds(row0 + r, CH)])
        pltpu.sync_copy(rows1.at[pl.ds(0, 65)],
                        acc.at[pl.ds(row0 + RPT - 65, 65)])

        @pl.loop(0, RPT - 65, step=CH)
        def _(r):
            pltpu.sync_copy(ones, cacc.at[pl.ds(row0 + r, CH)])
        pltpu.sync_copy(ones.at[pl.ds(0, 65)],
                        cacc.at[pl.ds(row0 + RPT - 65, 65)])

        @pl.loop(0, CH)
        def _(r):
            ones[r, pl.ds(0, CW)] = jnp.ones((CW,), jnp.float32)

        plsc.subcore_barrier()

        lane = lax.iota(jnp.int32, _LANES)

        def build_idx(j, b):
            base = j * CH
            for k in range(CH // _LANES):
                fvec = 2 * (base + k * _LANES + lane)
                s = plsc.load_gather(pbuf, [fvec])
                d = plsc.load_gather(pbuf, [fvec + 1])
                sidx[b][pl.ds(k * _LANES, _LANES)] = s + toff
                didx[b][pl.ds(k * _LANES, _LANES)] = d

        def start_gather(b):
            pltpu.async_copy(table_hbm.at[sidx[b]], rows[b], sems[b])

        def wait_gather(b):
            pltpu.make_async_copy(table_hbm.at[sidx[b]], rows[b],
                                  sems[b]).wait()

        def scatter(b):
            pltpu.async_copy(ones, cacc.at[didx[b]], semc, add=True)
            pltpu.sync_copy(rows[b], acc.at[didx[b]], add=True)
            pltpu.make_async_copy(ones, cacc.at[didx[b]], semc).wait()

        @pl.loop(0, NB)
        def _(blk):
            pltpu.sync_copy(
                pairs_hbm.at[pl.ds(2 * (wid * EPW + blk * EPB), 2 * EPB)],
                pbuf)

            build_idx(0, 0)
            start_gather(0)

            @pl.loop(0, (BCH - 1) // 2)
            def _(i):
                j = 2 * i
                build_idx(j + 1, 1)
                start_gather(1)
                wait_gather(0)
                scatter(0)
                build_idx(j + 2, 0)
                start_gather(0)
                wait_gather(1)
                scatter(1)

            wait_gather(0)
            scatter(0)

        plsc.subcore_barrier()

        pltpu.sync_copy(acc.at[pl.ds(row0, RPT)],
                        out_hbm.at[cid, pl.ds(row0, RPT)])
        pltpu.sync_copy(cacc.at[pl.ds(row0, RPT)],
                        cnt_hbm.at[cid, pl.ds(row0, RPT)])

    return run(pairs, table)


def _normalize(partials, counts, N, D, CW, BN):
    def body(p_ref, c_ref, o_ref):
        s = p_ref[0] + p_ref[1]
        cnt = (c_ref[0] + c_ref[1])[:, :1]
        div = jnp.where(cnt == 0.0, 1.0, cnt) + _SMALL
        o_ref[...] = s / div

    return pl.pallas_call(
        body,
        grid=(N // BN,),
        in_specs=[
            pl.BlockSpec((_NC, BN, D), lambda i: (0, i, 0)),
            pl.BlockSpec((_NC, BN, CW), lambda i: (0, i, 0)),
        ],
        out_specs=pl.BlockSpec((BN, D), lambda i: (i, 0)),
        out_shape=jax.ShapeDtypeStruct((N, D), jnp.float32),
    )(partials, counts)


def kernel(edge_lists, node_states, pos_lists, W, b):
    del pos_lists
    N, D = node_states.shape
    T, M, _ = edge_lists.shape
    E = T * M

    pairs = edge_lists.reshape(2 * E)
    Wt = W.T
    b2 = b.reshape(T, 1, D)

    table = _build_table(node_states, Wt, b2, T, N, D, BN=1000)
    partials, counts = _edge_scatter(table, pairs, N, D, E, M)
    return _normalize(partials, counts, N, D, CW=16, BN=1000)
